# baseline (speedup 1.0000x reference)
import functools

import jax
import jax.numpy as jnp
from jax.experimental import pallas as pl
from jax.experimental.pallas import tpu as pltpu

L = 1024
D_SEQ = 640
D_PAIR = 128
HID = 128
NL = 3
K = 16
NHP = 512
MIN_LOOP = 4

E = 2 * (L - 1) + 2 * L * K + 2 * NHP
B = 512
NB = (E + B - 1) // B
EP = NB * B


def _knn_kernel(c8_ref, cT_ref, nn_ref):
    iota_r = jax.lax.broadcasted_iota(jnp.int32, (L, L), 0)
    iota_c = jax.lax.broadcasted_iota(jnp.int32, (L, L), 1)
    d2 = jnp.zeros((L, L), jnp.float32)
    for c in range(3):
        dif = c8_ref[:, c:c + 1] - cT_ref[c:c + 1, :]
        d2 = d2 + dif * dif
    d2 = jnp.where(iota_r == iota_c, 1e18, d2)
    cols = []
    for _ in range(K):
        mn = jnp.min(d2, axis=1, keepdims=True)
        idx = jnp.min(jnp.where(d2 == mn, iota_c, jnp.int32(2**30)),
                      axis=1, keepdims=True)
        cols.append(idx)
        d2 = jnp.where(iota_c == idx, 1e18, d2)
    nn_ref[:] = jnp.concatenate(cols, axis=1)


def _knn(coords8):
    return pl.pallas_call(
        _knn_kernel,
        out_shape=jax.ShapeDtypeStruct((L, K), jnp.int32),
    )(coords8, coords8.T)


def _build_edges(coords8, bppm):
    i = jnp.arange(L - 1)
    bb_src = jnp.concatenate([i, i + 1])
    bb_dst = jnp.concatenate([i + 1, i])
    nn_idx = _knn(coords8)
    knn_src = jnp.repeat(jnp.arange(L), K)
    knn_dst = nn_idx.reshape(-1)
    triu = jnp.triu(bppm, k=MIN_LOOP + 1)
    _, top_idx = jax.lax.top_k(triu.reshape(-1), NHP)
    hp_i = top_idx // L
    hp_j = top_idx % L
    src = jnp.concatenate([bb_src, knn_src, knn_dst, hp_i, hp_j])
    dst = jnp.concatenate([bb_dst, knn_dst, knn_src, hp_j, hp_i])
    return src, dst


def _mp_kernel(seq_ref, coords_ref, src_c_ref, src_r_ref, dst_c_ref, dst_r_ref,
               pe_ref, bp_ref, W_in_ref, W1a_ref, W1b_ref, W1c_ref, wbp_ref,
               wds_ref, b1_ref, W2_ref, Wha_ref, Whb_ref, wc_ref,
               out_ref, h_ref, x_ref, acc1_ref, acc2_ref):
    l = pl.program_id(0)
    b = pl.program_id(1)

    @pl.when(jnp.logical_and(l == 0, b == 0))
    def _init():
        h_ref[:] = jnp.dot(seq_ref[:], W_in_ref[:],
                           preferred_element_type=jnp.float32)
        x_ref[:] = coords_ref[:]

    @pl.when(jnp.logical_and(l > 0, b == 0))
    def _node_update():
        lm1 = l - 1
        hh = h_ref[:]
        agg = acc1_ref[:]
        upd = acc2_ref[:, :3]
        deg = acc2_ref[:, 3:4]
        h_ref[:] = hh + jax.nn.relu(
            jnp.dot(hh, Wha_ref[lm1], preferred_element_type=jnp.float32)
            + jnp.dot(agg, Whb_ref[lm1], preferred_element_type=jnp.float32))
        x_ref[:, :3] = x_ref[:, :3] + upd / (deg + 1.0)

    @pl.when(b == 0)
    def _reset():
        acc1_ref[:] = jnp.zeros_like(acc1_ref)
        acc2_ref[:] = jnp.zeros_like(acc2_ref)

    src_c = src_c_ref[0]
    dst_c = dst_c_ref[0]
    dst_r = dst_r_ref[0]
    bp = bp_ref[0]

    iota_bl = jax.lax.broadcasted_iota(jnp.int32, (B, L), 1)
    eidx_c = jax.lax.broadcasted_iota(jnp.int32, (B, 1), 0) + b * B
    valid_c = eidx_c < E
    oh_s = jnp.where((iota_bl == src_c) & valid_c, 1.0, 0.0)
    oh_d = jnp.where((iota_bl == dst_c) & valid_c, 1.0, 0.0)
    iota_lb = jax.lax.broadcasted_iota(jnp.int32, (L, B), 0)
    valid_r = (jax.lax.broadcasted_iota(jnp.int32, (1, B), 1) + b * B) < E
    oh_dT = jnp.where((iota_lb == dst_r) & valid_r, 1.0, 0.0)

    h = h_ref[:].astype(jnp.bfloat16)
    x = x_ref[:]
    oh_s_bf = oh_s.astype(jnp.bfloat16)
    oh_d_bf = oh_d.astype(jnp.bfloat16)
    h_s = jnp.dot(oh_s_bf, h, preferred_element_type=jnp.float32)
    h_d = jnp.dot(oh_d_bf, h, preferred_element_type=jnp.float32)
    rel = jnp.dot(oh_s - oh_d, x, preferred_element_type=jnp.float32)
    dist = jnp.sqrt(jnp.sum(rel * rel, axis=1, keepdims=True) + 1e-12)

    pre = (jnp.dot(h_s.astype(jnp.bfloat16), W1a_ref[l],
                   preferred_element_type=jnp.float32)
           + jnp.dot(h_d.astype(jnp.bfloat16), W1b_ref[l],
                     preferred_element_type=jnp.float32)
           + jnp.dot(pe_ref[:].astype(jnp.bfloat16), W1c_ref[l],
                     preferred_element_type=jnp.float32)
           + bp * wbp_ref[l]
           + dist * wds_ref[l]
           + b1_ref[l])
    hdn = jax.nn.relu(pre).astype(jnp.bfloat16)
    m = jnp.dot(hdn, W2_ref[l], preferred_element_type=jnp.float32)
    wgt = jnp.tanh(jnp.sum(m * wc_ref[l], axis=1, keepdims=True))
    relw = rel * wgt
    lane8 = jax.lax.broadcasted_iota(jnp.int32, (B, 8), 1)
    payload2 = jnp.where(lane8 == 3, 1.0, relw)

    acc1_ref[:] += jnp.dot(oh_dT.astype(jnp.bfloat16), m.astype(jnp.bfloat16),
                           preferred_element_type=jnp.float32)
    acc2_ref[:] += jnp.dot(oh_dT, payload2, preferred_element_type=jnp.float32)

    @pl.when(jnp.logical_and(l == NL - 1, b == NB - 1))
    def _final():
        upd = acc2_ref[:, :3]
        deg = acc2_ref[:, 3:4]
        out_ref[:] = x_ref[:, :3] + upd / (deg + 1.0)


def kernel(seq_embed, pair_embed, bppm, coords, W_in, W1, b1, W2, Wh, Wc):
    coords8 = jnp.pad(coords, ((0, 0), (0, 5)))
    src, dst = _build_edges(coords8, bppm)
    src = jnp.concatenate([src, jnp.zeros((EP - E,), jnp.int32)]).astype(jnp.int32)
    dst = jnp.concatenate([dst, jnp.zeros((EP - E,), jnp.int32)]).astype(jnp.int32)

    edge_pe = pair_embed[src, dst]
    edge_b = bppm[src, dst]

    src_c = src.reshape(NB, B, 1)
    dst_c = dst.reshape(NB, B, 1)
    src_r = src.reshape(NB, 1, B)
    dst_r = dst.reshape(NB, 1, B)
    bp_c = edge_b.reshape(NB, B, 1)

    W1a = W1[:, :HID, :].astype(jnp.bfloat16)
    W1b = W1[:, HID:2 * HID, :].astype(jnp.bfloat16)
    W1c = W1[:, 2 * HID:2 * HID + D_PAIR, :].astype(jnp.bfloat16)
    wbp = W1[:, 2 * HID + D_PAIR, :]
    wds = W1[:, 2 * HID + D_PAIR + 1, :]
    W2 = W2.astype(jnp.bfloat16)
    Wha = Wh[:, :HID, :]
    Whb = Wh[:, HID:, :]
    wc = Wc[:, :, 0]

    grid = (NL, NB)
    full = lambda shape: pl.BlockSpec(shape, lambda l, b: tuple(0 for _ in shape))
    eblk3 = lambda shape: pl.BlockSpec(shape, lambda l, b: (b, 0, 0))

    out = pl.pallas_call(
        _mp_kernel,
        grid=grid,
        in_specs=[
            full((L, D_SEQ)),
            full((L, 8)),
            eblk3((1, B, 1)),
            eblk3((1, 1, B)),
            eblk3((1, B, 1)),
            eblk3((1, 1, B)),
            pl.BlockSpec((B, D_PAIR), lambda l, b: (b, 0)),
            eblk3((1, B, 1)),
            full((D_SEQ, HID)),
            full((NL, HID, 256)),
            full((NL, HID, 256)),
            full((NL, D_PAIR, 256)),
            full((NL, 256)),
            full((NL, 256)),
            full((NL, 256)),
            full((NL, 256, HID)),
            full((NL, HID, HID)),
            full((NL, HID, HID)),
            full((NL, HID)),
        ],
        out_specs=pl.BlockSpec((L, 3), lambda l, b: (0, 0)),
        out_shape=jax.ShapeDtypeStruct((L, 3), jnp.float32),
        scratch_shapes=[
            pltpu.VMEM((L, HID), jnp.float32),
            pltpu.VMEM((L, 8), jnp.float32),
            pltpu.VMEM((L, HID), jnp.float32),
            pltpu.VMEM((L, 8), jnp.float32),
        ],
    )(seq_embed, coords8, src_c, src_r, dst_c, dst_r, edge_pe, bp_c,
      W_in, W1a, W1b, W1c, wbp, wds, b1, W2, Wha, Whb, wc)
    return out


# speedup vs baseline: 2.1726x; 2.1726x over previous
import functools

import jax
from jax import lax
import jax.numpy as jnp
from jax.experimental import pallas as pl
from jax.experimental.pallas import tpu as pltpu
from jax.experimental.pallas import tpu_sc as plsc

L = 1024
D_SEQ = 640
D_PAIR = 128
HID = 128
NL = 3
K = 16
NHP = 512
MIN_LOOP = 4

NC = 2
NS = 16
NW = NC * NS
LN = 16
RPW = L // NW
SLAB = RPW * L
STG = 512
SENT = L * L
HP_CAP = 1024

E = 2 * (L - 1) + 2 * L * K + 2 * HP_CAP
B = 512
NB = (E + B - 1) // B
EP = NB * B


def _knn_thresh_kernel(c8_ref, cT_ref, bppm_ref, nn_ref, t_ref, vm_ref):
    iota_r2 = jax.lax.broadcasted_iota(jnp.int32, (L, L), 0)
    iota_c2 = jax.lax.broadcasted_iota(jnp.int32, (L, L), 1)
    vm_ref[:] = jnp.where(iota_c2 - iota_r2 >= MIN_LOOP + 1, bppm_ref[:], -1.0)

    def bs_body(_, lohi):
        lo, hi = lohi
        mid = 0.5 * (lo + hi)
        cnt = jnp.sum(jnp.where(vm_ref[:] >= mid, 1.0, 0.0))
        ge = cnt >= float(NHP)
        return (jnp.where(ge, mid, lo), jnp.where(ge, hi, mid))

    lo, _ = jax.lax.fori_loop(0, 50, bs_body,
                              (jnp.float32(-2.0), jnp.float32(2.0)))
    t_ref[:] = jnp.full((8, 128), lo, jnp.float32)
    _knn_body(c8_ref, cT_ref, nn_ref)


def _knn_body(c8_ref, cT_ref, nn_ref):
    iota_r = jax.lax.broadcasted_iota(jnp.int32, (L, L), 0)
    iota_c = jax.lax.broadcasted_iota(jnp.int32, (L, L), 1)
    d2 = jnp.zeros((L, L), jnp.float32)
    for c in range(3):
        dif = c8_ref[:, c:c + 1] - cT_ref[c:c + 1, :]
        d2 = d2 + dif * dif
    d2 = jnp.where(iota_r == iota_c, 1e18, d2)
    cols = []
    for _ in range(K):
        mn = jnp.min(d2, axis=1, keepdims=True)
        idx = jnp.min(jnp.where(d2 == mn, iota_c, jnp.int32(2**30)),
                      axis=1, keepdims=True)
        cols.append(idx)
        d2 = jnp.where(iota_c == idx, 1e18, d2)
    nn_ref[:] = jnp.concatenate(cols, axis=1)


def _knn_thresh(coords8, bppm):
    return pl.pallas_call(
        _knn_thresh_kernel,
        out_shape=(jax.ShapeDtypeStruct((L, K), jnp.int32),
                   jax.ShapeDtypeStruct((8, 128), jnp.float32)),
        scratch_shapes=[pltpu.VMEM((L, L), jnp.float32)],
    )(coords8, coords8.T, bppm)


_SC_MESH = plsc.VectorSubcoreMesh(core_axis_name="c", subcore_axis_name="s",
                                  num_cores=NC, num_subcores=NS)

def _sc_params():
    cp = pltpu.CompilerParams()
    if "needs_layout_passes" in pltpu.CompilerParams.__dataclass_fields__:
        import dataclasses
        cp = dataclasses.replace(cp, needs_layout_passes=False)
    return cp


def _wid():
    return lax.axis_index("s") * NC + lax.axis_index("c")


def _lane_extract(vec, lane):
    return jnp.sum(jnp.where(
        jax.lax.broadcasted_iota(jnp.int32, (LN,), 0) == lane, vec, 0))


def _hp_scan_body(bppm_hbm, t_hbm, cnt_hbm, gts_hbm, eqs_hbm,
                  slab_v, t_v, cnt_v, gtb_v, eqb_v):
    wid = _wid()
    base_row = wid * RPW
    pltpu.sync_copy(bppm_hbm.at[pl.ds(wid * SLAB, SLAB)], slab_v)
    pltpu.sync_copy(t_hbm, t_v)
    t = t_v[...]
    iota = jax.lax.broadcasted_iota(jnp.int32, (LN,), 0)

    def row_body(r, carry):
        row = base_row + r
        cmin = jnp.maximum((row - 10) // LN, 0)

        def col_body(c, carry2):
            gtc, eqc = carry2
            off = r * L + c * LN
            v = slab_v[pl.ds(off, LN)]
            col = c * LN + iota
            tri = (col - row) >= (MIN_LOOP + 1)
            m_gt = jnp.logical_and(tri, v > t)
            m_eq = jnp.logical_and(tri, v == t)
            flat = wid * SLAB + off + iota
            pos_gt = gtc + plsc.cumsum(m_gt.astype(jnp.int32)) - 1
            plsc.store_scatter(gtb_v, [pos_gt], flat,
                               mask=jnp.logical_and(m_gt, pos_gt < STG))
            pos_eq = eqc + plsc.cumsum(m_eq.astype(jnp.int32)) - 1
            plsc.store_scatter(eqb_v, [pos_eq], flat,
                               mask=jnp.logical_and(m_eq, pos_eq < STG))
            ng = jnp.sum(m_gt.astype(jnp.int32))
            ne = jnp.sum(m_eq.astype(jnp.int32))
            return (gtc + ng, eqc + ne)

        return lax.fori_loop(cmin, L // LN, col_body, carry)

    gtc, eqc = lax.fori_loop(0, RPW, row_body,
                             (jnp.int32(0), jnp.int32(0)))
    cnt_v[...] = (jnp.where(iota == 0, gtc, 0)
                  + jnp.where(iota == 1, eqc, 0))
    pltpu.sync_copy(cnt_v, cnt_hbm.at[pl.ds(wid * LN, LN)])
    pltpu.sync_copy(gtb_v, gts_hbm.at[pl.ds(wid * STG, STG)])
    pltpu.sync_copy(eqb_v, eqs_hbm.at[pl.ds(wid * STG, STG)])


def _hp_emit_body(cnt_hbm, gts_hbm, eqs_hbm, out_hbm, dbg_hbm,
                  cnts_v, sgt_v, seq_v, buf_v, dbg_v):
    wid = _wid()
    iota = jax.lax.broadcasted_iota(jnp.int32, (LN,), 0)
    pltpu.sync_copy(cnt_hbm, cnts_v)

    def tot_body(w, carry):
        c1, eqt = carry
        row = cnts_v[pl.ds(w * LN, LN)]
        return (c1 + _lane_extract(row, 0), eqt + _lane_extract(row, 1))

    c1, _ = lax.fori_loop(0, NW, tot_body, (jnp.int32(0), jnp.int32(0)))
    need_eq = NHP - c1

    def pre_body(w, carry):
        eqpre, base, myg, myq = carry
        row = cnts_v[pl.ds(w * LN, LN)]
        g = _lane_extract(row, 0)
        e = _lane_extract(row, 1)
        quota = jnp.clip(need_eq - eqpre, 0, e)
        sel = g + quota
        pad = ((sel + LN - 1) // LN) * LN
        return (eqpre + e,
                jnp.where(w < wid, base + pad, base),
                jnp.where(w == wid, g, myg),
                jnp.where(w == wid, quota, myq))

    _, base, myg, myq = lax.fori_loop(
        0, NW, pre_body,
        (jnp.int32(0), jnp.int32(0), jnp.int32(0), jnp.int32(0)))
    base = pl.multiple_of(base, LN)

    dbg_v[...] = (jnp.where(iota == 0, c1, 0) + jnp.where(iota == 1, base, 0)
                  + jnp.where(iota == 2, myg, 0) + jnp.where(iota == 3, myq, 0))
    pltpu.sync_copy(dbg_v, dbg_hbm.at[pl.ds(wid * LN, LN)])

    sel = myg + myq

    sentv = jnp.full((LN,), SENT, jnp.int32)
    for i in range(HP_CAP // LN):
        buf_v[pl.ds(i * LN, LN)] = sentv

    pltpu.sync_copy(gts_hbm.at[pl.ds(wid * STG, STG)], sgt_v)
    pltpu.sync_copy(eqs_hbm.at[pl.ds(wid * STG, STG)], seq_v)

    def g_body(j, _):
        vals = sgt_v[pl.ds(j * LN, LN)]
        pos = j * LN + iota
        plsc.store_scatter(buf_v, [pos], vals, mask=pos < myg)
        return 0

    lax.fori_loop(0, (myg + LN - 1) // LN, g_body, 0)

    def e_body(j, _):
        vals = seq_v[pl.ds(j * LN, LN)]
        rank = j * LN + iota
        plsc.store_scatter(buf_v, [myg + rank], vals, mask=rank < myq)
        return 0

    lax.fori_loop(0, (myq + LN - 1) // LN, e_body, 0)

    nch = jnp.where(wid == NW - 1, (HP_CAP - base) // LN,
                    (sel + LN - 1) // LN)

    def o_body(j, _):
        pltpu.sync_copy(buf_v.at[pl.ds(j * LN, LN)],
                        out_hbm.at[pl.ds(base + j * LN, LN)])
        return 0

    lax.fori_loop(0, nch, o_body, 0)


def _hp_select(bppm_flat, t16):
    scan = pl.kernel(
        _hp_scan_body,
        out_type=(jax.ShapeDtypeStruct((NW * LN,), jnp.int32),
                  jax.ShapeDtypeStruct((NW * STG,), jnp.int32),
                  jax.ShapeDtypeStruct((NW * STG,), jnp.int32)),
        mesh=_SC_MESH,
        compiler_params=_sc_params(),
        scratch_types=[pltpu.VMEM((SLAB,), jnp.float32),
                       pltpu.VMEM((LN,), jnp.float32),
                       pltpu.VMEM((LN,), jnp.int32),
                       pltpu.VMEM((STG,), jnp.int32),
                       pltpu.VMEM((STG,), jnp.int32)])
    cnts, sgt, seq = scan(bppm_flat, t16)
    emit = pl.kernel(
        _hp_emit_body,
        out_type=(jax.ShapeDtypeStruct((HP_CAP,), jnp.int32),
                  jax.ShapeDtypeStruct((NW * LN,), jnp.int32)),
        mesh=_SC_MESH,
        compiler_params=_sc_params(),
        scratch_types=[pltpu.VMEM((NW * LN,), jnp.int32),
                       pltpu.VMEM((STG,), jnp.int32),
                       pltpu.VMEM((STG,), jnp.int32),
                       pltpu.VMEM((HP_CAP,), jnp.int32),
                       pltpu.VMEM((LN,), jnp.int32)])
    hp, _ = emit(cnts, sgt, seq)
    return hp


def _build_edges(coords8, bppm):
    i = jnp.arange(L - 1)
    bb_src = jnp.concatenate([i, i + 1])
    bb_dst = jnp.concatenate([i + 1, i])
    nn_idx, t_full = _knn_thresh(coords8, bppm)
    knn_src = jnp.repeat(jnp.arange(L), K)
    knn_dst = nn_idx.reshape(-1)
    hp = _hp_select(bppm.reshape(-1), t_full.reshape(-1)[:LN])
    hp_i = hp // L
    hp_j = hp % L
    src = jnp.concatenate([bb_src, knn_src, knn_dst, hp_i, hp_j])
    dst = jnp.concatenate([bb_dst, knn_dst, knn_src, hp_j, hp_i])
    return src, dst


def _mp_kernel(seq_ref, coords_ref, src_c_ref, src_r_ref, dst_c_ref, dst_r_ref,
               pe_ref, bp_ref, W_in_ref, W1a_ref, W1b_ref, W1c_ref, wbp_ref,
               wds_ref, b1_ref, W2_ref, Wha_ref, Whb_ref, wc_ref,
               out_ref, h_ref, x_ref, acc1_ref, acc2_ref):
    l = pl.program_id(0)
    b = pl.program_id(1)

    @pl.when(jnp.logical_and(l == 0, b == 0))
    def _init():
        h_ref[:] = jnp.dot(seq_ref[:], W_in_ref[:],
                           preferred_element_type=jnp.float32)
        x_ref[:] = coords_ref[:]

    @pl.when(jnp.logical_and(l > 0, b == 0))
    def _node_update():
        lm1 = l - 1
        hh = h_ref[:]
        agg = acc1_ref[:]
        upd = acc2_ref[:, :3]
        deg = acc2_ref[:, 3:4]
        h_ref[:] = hh + jax.nn.relu(
            jnp.dot(hh, Wha_ref[lm1], preferred_element_type=jnp.float32)
            + jnp.dot(agg, Whb_ref[lm1], preferred_element_type=jnp.float32))
        x_ref[:, :3] = x_ref[:, :3] + upd / (deg + 1.0)

    @pl.when(b == 0)
    def _reset():
        acc1_ref[:] = jnp.zeros_like(acc1_ref)
        acc2_ref[:] = jnp.zeros_like(acc2_ref)

    src_c = src_c_ref[0]
    dst_c = dst_c_ref[0]
    src_r = src_r_ref[0]
    dst_r = dst_r_ref[0]
    bp = bp_ref[0]

    iota_bl = jax.lax.broadcasted_iota(jnp.int32, (B, L), 1)
    oh_s = jnp.where(iota_bl == src_c, 1.0, 0.0)
    oh_d = jnp.where(iota_bl == dst_c, 1.0, 0.0)
    iota_lb = jax.lax.broadcasted_iota(jnp.int32, (L, B), 0)
    oh_dT = jnp.where((iota_lb == dst_r) & (src_r < L), 1.0, 0.0)

    h = h_ref[:].astype(jnp.bfloat16)
    x = x_ref[:]
    oh_s_bf = oh_s.astype(jnp.bfloat16)
    oh_d_bf = oh_d.astype(jnp.bfloat16)
    h_s = jnp.dot(oh_s_bf, h, preferred_element_type=jnp.float32)
    h_d = jnp.dot(oh_d_bf, h, preferred_element_type=jnp.float32)
    rel = jnp.dot(oh_s - oh_d, x, preferred_element_type=jnp.float32)
    dist = jnp.sqrt(jnp.sum(rel * rel, axis=1, keepdims=True) + 1e-12)

    pre = (jnp.dot(h_s.astype(jnp.bfloat16), W1a_ref[l],
                   preferred_element_type=jnp.float32)
           + jnp.dot(h_d.astype(jnp.bfloat16), W1b_ref[l],
                     preferred_element_type=jnp.float32)
           + jnp.dot(pe_ref[:].astype(jnp.bfloat16), W1c_ref[l],
                     preferred_element_type=jnp.float32)
           + bp * wbp_ref[l]
           + dist * wds_ref[l]
           + b1_ref[l])
    hdn = jax.nn.relu(pre).astype(jnp.bfloat16)
    m = jnp.dot(hdn, W2_ref[l], preferred_element_type=jnp.float32)
    wgt = jnp.tanh(jnp.sum(m * wc_ref[l], axis=1, keepdims=True))
    relw = rel * wgt
    lane8 = jax.lax.broadcasted_iota(jnp.int32, (B, 8), 1)
    payload2 = jnp.where(lane8 == 3, 1.0, relw)

    acc1_ref[:] += jnp.dot(oh_dT.astype(jnp.bfloat16), m.astype(jnp.bfloat16),
                           preferred_element_type=jnp.float32)
    acc2_ref[:] += jnp.dot(oh_dT, payload2, preferred_element_type=jnp.float32)

    @pl.when(jnp.logical_and(l == NL - 1, b == NB - 1))
    def _final():
        upd = acc2_ref[:, :3]
        deg = acc2_ref[:, 3:4]
        out_ref[:] = x_ref[:, :3] + upd / (deg + 1.0)


def kernel(seq_embed, pair_embed, bppm, coords, W_in, W1, b1, W2, Wh, Wc):
    coords8 = jnp.pad(coords, ((0, 0), (0, 5)))
    src, dst = _build_edges(coords8, bppm)
    src = jnp.concatenate([src, jnp.full((EP - E,), L)]).astype(jnp.int32)
    dst = jnp.concatenate([dst, jnp.full((EP - E,), L)]).astype(jnp.int32)

    edge_pe = pair_embed[src, dst]
    edge_b = bppm[src, dst]

    src_c = src.reshape(NB, B, 1)
    dst_c = dst.reshape(NB, B, 1)
    src_r = src.reshape(NB, 1, B)
    dst_r = dst.reshape(NB, 1, B)
    bp_c = edge_b.reshape(NB, B, 1)

    W1a = W1[:, :HID, :].astype(jnp.bfloat16)
    W1b = W1[:, HID:2 * HID, :].astype(jnp.bfloat16)
    W1c = W1[:, 2 * HID:2 * HID + D_PAIR, :].astype(jnp.bfloat16)
    wbp = W1[:, 2 * HID + D_PAIR, :]
    wds = W1[:, 2 * HID + D_PAIR + 1, :]
    W2 = W2.astype(jnp.bfloat16)
    Wha = Wh[:, :HID, :]
    Whb = Wh[:, HID:, :]
    wc = Wc[:, :, 0]

    grid = (NL, NB)
    full = lambda shape: pl.BlockSpec(shape, lambda l, b: tuple(0 for _ in shape))
    eblk3 = lambda shape: pl.BlockSpec(shape, lambda l, b: (b, 0, 0))

    out = pl.pallas_call(
        _mp_kernel,
        grid=grid,
        in_specs=[
            full((L, D_SEQ)),
            full((L, 8)),
            eblk3((1, B, 1)),
            eblk3((1, 1, B)),
            eblk3((1, B, 1)),
            eblk3((1, 1, B)),
            pl.BlockSpec((B, D_PAIR), lambda l, b: (b, 0)),
            eblk3((1, B, 1)),
            full((D_SEQ, HID)),
            full((NL, HID, 256)),
            full((NL, HID, 256)),
            full((NL, D_PAIR, 256)),
            full((NL, 256)),
            full((NL, 256)),
            full((NL, 256)),
            full((NL, 256, HID)),
            full((NL, HID, HID)),
            full((NL, HID, HID)),
            full((NL, HID)),
        ],
        out_specs=pl.BlockSpec((L, 3), lambda l, b: (0, 0)),
        out_shape=jax.ShapeDtypeStruct((L, 3), jnp.float32),
        scratch_shapes=[
            pltpu.VMEM((L, HID), jnp.float32),
            pltpu.VMEM((L, 8), jnp.float32),
            pltpu.VMEM((L, HID), jnp.float32),
            pltpu.VMEM((L, 8), jnp.float32),
        ],
    )(seq_embed, coords8, src_c, src_r, dst_c, dst_r, edge_pe, bp_c,
      W_in, W1a, W1b, W1c, wbp, wds, b1, W2, Wha, Whb, wc)
    return out
